# baseline (speedup 1.0000x reference)
import jax
import jax.numpy as jnp
from jax.experimental import pallas as pl

_E = 16


def _diag_kernel(x_ref, out_ref):
    out_ref[0] = x_ref[0, :1, :16, 0]


def kernel(x, W, b):
    B = x.shape[0]
    out = pl.pallas_call(
        _diag_kernel,
        grid=(B,),
        in_specs=[
            pl.BlockSpec((1, 768, 24, 24), lambda i: (i, 0, 0, 0)),
        ],
        out_specs=pl.BlockSpec((1, 1, _E), lambda i: (i, 0, 0)),
        out_shape=jax.ShapeDtypeStruct((B, 1, _E), jnp.float32),
    )(x)
    return out.reshape(B, _E)


# speedup vs baseline: 1.0951x; 1.0951x over previous
import numpy as np
import jax
import jax.numpy as jnp
from jax.experimental import pallas as pl

_E = 16
_CG, _CI = 24, 32
_HG, _HI = 3, 8
_WG, _WI = 6, 4

_G_NP = (np.arange(128)[:, None] // 4 == np.arange(512)[None, :] // 16
         ).astype(np.float32)
_F2_NP = (np.arange(512)[:, None] % 16 == np.arange(16)[None, :]
          ).astype(np.float32)


def _gate_kernel(x_ref, g_ref, wr_ref, f2_ref, b_ref, out_ref):
    xt = x_ref[0]
    t = jnp.sum(xt, axis=(1, 2, 3))
    v = jnp.dot(t, g_ref[...], precision=jax.lax.Precision.HIGHEST,
                preferred_element_type=jnp.float32)
    p = v * wr_ref[...]
    q = jnp.sum(p, axis=0, keepdims=True)
    logits = jnp.dot(q, f2_ref[...], precision=jax.lax.Precision.HIGHEST,
                     preferred_element_type=jnp.float32) * (1.0 / 576.0)
    logits = logits + b_ref[...]
    iota = jax.lax.broadcasted_iota(jnp.int32, (1, _E), 1)
    m = jnp.max(logits, axis=1, keepdims=True)
    first = jnp.min(jnp.where(logits == m, iota, _E), axis=1, keepdims=True)
    out_ref[0] = (iota == first).astype(jnp.float32)


def kernel(x, W, b):
    B = x.shape[0]
    x6 = (x.reshape(B, _CG, _CI, _HG, _HI, _WG, _WI)
           .transpose(0, 1, 3, 5, 4, 2, 6)
           .reshape(B, _CG, _HG, _WG, _HI, _CI * _WI))
    wr = W.T.reshape(_CG, _CI * _E)
    b2 = b.reshape(1, _E)
    g = jnp.asarray(_G_NP)
    f2 = jnp.asarray(_F2_NP)
    out = pl.pallas_call(
        _gate_kernel,
        grid=(B,),
        in_specs=[
            pl.BlockSpec((1, _CG, _HG, _WG, _HI, 128),
                         lambda i: (i, 0, 0, 0, 0, 0)),
            pl.BlockSpec((128, 512), lambda i: (0, 0)),
            pl.BlockSpec((_CG, 512), lambda i: (0, 0)),
            pl.BlockSpec((512, _E), lambda i: (0, 0)),
            pl.BlockSpec((1, _E), lambda i: (0, 0)),
        ],
        out_specs=pl.BlockSpec((1, 1, _E), lambda i: (i, 0, 0)),
        out_shape=jax.ShapeDtypeStruct((B, 1, _E), jnp.float32),
    )(x6, g, wr, f2, b2)
    return out.reshape(B, _E)
